# Initial kernel scaffold; baseline (speedup 1.0000x reference)
import jax
import jax.numpy as jnp
from jax.experimental import pallas as pl


def kernel(x):
    raise NotImplementedError("write your pallas kernel here")


# speedup vs baseline: 3.5602x; 3.5602x over previous
import functools

import jax
import jax.numpy as jnp
import numpy as np
from jax import lax
from jax.experimental import pallas as pl

_K = 3
_TINY = np.float32(np.finfo(np.float32).tiny)
_ROT_A = (13, 15, 26, 6)
_ROT_B = (17, 29, 16, 24)


def _threefry_rounds(x0, x1, rots):
    for r in rots:
        x0 = x0 + x1
        x1 = lax.shift_left(x1, np.int32(r)) | lax.shift_right_logical(
            x1, np.int32(32 - r)
        )
        x1 = x1 ^ x0
    return x0, x1


def _gumbel_from_index(idx):
    ks0 = np.int32(0)
    ks1 = np.int32(42)
    ks2 = np.int32(0x1BD11BDA ^ 42)
    x0 = jnp.zeros_like(idx) + ks0
    x1 = idx + ks1
    x0, x1 = _threefry_rounds(x0, x1, _ROT_A)
    x0, x1 = x0 + ks1, x1 + ks2 + np.int32(1)
    x0, x1 = _threefry_rounds(x0, x1, _ROT_B)
    x0, x1 = x0 + ks2, x1 + ks0 + np.int32(2)
    x0, x1 = _threefry_rounds(x0, x1, _ROT_A)
    x0, x1 = x0 + ks0, x1 + ks1 + np.int32(3)
    x0, x1 = _threefry_rounds(x0, x1, _ROT_B)
    x0, x1 = x0 + ks1, x1 + ks2 + np.int32(4)
    x0, x1 = _threefry_rounds(x0, x1, _ROT_A)
    x0, x1 = x0 + ks2, x1 + ks0 + np.int32(5)
    bits = x0 ^ x1
    float_bits = lax.shift_right_logical(bits, np.int32(9)) | np.int32(0x3F800000)
    f = lax.bitcast_convert_type(float_bits, jnp.float32) - np.float32(1.0)
    u = jnp.maximum(_TINY, f * (np.float32(1.0) - _TINY) + _TINY)
    return -jnp.log(-jnp.log(u))


def _pool_kernel(x_ref, o_ref, *, oh, ow):
    c = pl.program_id(0)
    xb = x_ref[0]

    denom = None
    for dy in range(_K):
        for dx in range(_K):
            r = jnp.maximum(xb[dy : dy + oh, dx : dx + ow], np.float32(0.0))
            denom = r if denom is None else denom + r
    zero_den = denom == np.float32(0.0)

    y = lax.broadcasted_iota(jnp.int32, (oh, ow), 0)
    xx = lax.broadcasted_iota(jnp.int32, (oh, ow), 1)
    base = ((c * oh + y) * ow + xx) * np.int32(9)

    best_score = jnp.full((oh, ow), -jnp.inf, jnp.float32)
    best_val = jnp.zeros((oh, ow), jnp.float32)
    for j in range(9):
        dy, dx = divmod(j, _K)
        p = xb[dy : dy + oh, dx : dx + ow]
        g = _gumbel_from_index(base + np.int32(j))
        prob = jnp.where(zero_den, np.float32(1.0), jnp.maximum(p, 0.0) / denom)
        score = g + jnp.log(prob)
        take = score > best_score
        best_score = jnp.where(take, score, best_score)
        best_val = jnp.where(take, p, best_val)
    o_ref[0] = best_val


@jax.jit
def kernel(x):
    B, C, H, W = x.shape
    oh = H - _K + 1
    ow = W - _K + 1
    x3 = x.reshape(B * C, H, W)
    out = pl.pallas_call(
        functools.partial(_pool_kernel, oh=oh, ow=ow),
        grid=(B * C,),
        in_specs=[pl.BlockSpec((1, H, W), lambda c: (c, 0, 0))],
        out_specs=pl.BlockSpec((1, oh, ow), lambda c: (c, 0, 0)),
        out_shape=jax.ShapeDtypeStruct((B * C, oh, ow), jnp.float32),
    )(x3)
    return out.reshape(B, C, oh, ow)
